# baseline (speedup 1.0000x reference)
import functools

import jax
import jax.numpy as jnp
from jax import lax
from jax.experimental import pallas as pl
from jax.experimental.pallas import tpu as pltpu
from jax.experimental.pallas import tpu_sc as plsc

_B = 16384
_D = 100
_DP = 128
_N = 1000000
_NC = 2
_NS = 16
_NW = _NC * _NS
_BPW = _B // _NW
_SPT = _B // _NS
_ES = 10
_MOM = 0.9


def _sc_scatter_body(idx_hbm, pos_hbm, post, sidx2, spos2, sem_s):
    c = lax.axis_index("c")
    s = lax.axis_index("s")
    wid = s * _NC + c
    base = wid * _BPW
    for k in range(_BPW // 128):
        pltpu.sync_copy(idx_hbm.at[pl.ds(base + k * 128, 128)],
                        sidx2.at[k])
        pltpu.sync_copy(pos_hbm.at[pl.ds(base + k * 128, 128)],
                        spos2.at[k])
    scps = [
        pltpu.async_copy(spos2.at[k], post.at[sidx2.at[k]], sem_s)
        for k in range(_BPW // 128)
    ]
    for cp in scps:
        cp.wait()


def _sc_scatter(index, pos):
    fn = pl.kernel(
        _sc_scatter_body,
        out_type=jax.ShapeDtypeStruct((_N,), jnp.int32),
        mesh=plsc.VectorSubcoreMesh(
            core_axis_name="c", subcore_axis_name="s",
            num_cores=_NC, num_subcores=_NS,
        ),
        scratch_types=[
            pltpu.VMEM((_BPW // 128, 128), jnp.int32),
            pltpu.VMEM((_BPW // 128, 128), jnp.int32),
            pltpu.SemaphoreType.DMA,
        ],
    )
    return fn(index, pos)


def _sc_body(slp, lgp, idx_hbm, post,
             g_out, l2_out,
             idx_v, oidx2, w2, w_v, rowbuf,
             sem_s, sem_g):
    c = lax.axis_index("c")
    s = lax.axis_index("s")
    wid = s * _NC + c
    base = wid * _BPW

    pltpu.sync_copy(idx_hbm.at[pl.ds(base, _BPW)], idx_v)

    gcp = pltpu.async_copy(slp.at[idx_v], rowbuf, sem_g)

    for k in range(_BPW // 128):
        pltpu.sync_copy(idx_hbm.at[pl.ds(base + k * 128, 128)],
                        oidx2.at[k])
    wcps = [
        pltpu.async_copy(post.at[oidx2.at[k]], w2.at[k], sem_s)
        for k in range(_BPW // 128)
    ]
    for cp in wcps:
        cp.wait()
    for k in range(_BPW // 128):
        for j in range(8):
            v = w2[k, pl.ds(j * 16, 16)]
            w_v[pl.ds(k * 128 + j * 16, 16)] = jnp.clip(v, 0, _B - 1)

    gcp.wait()
    pltpu.sync_copy(rowbuf, g_out.at[pl.ds(base, _BPW)])

    pltpu.async_copy(lgp.at[w_v], rowbuf, sem_g).wait()
    pltpu.sync_copy(rowbuf, l2_out.at[pl.ds(base, _BPW)])


def _sc_gather(slp, lgp, index, post):
    fn = pl.kernel(
        _sc_body,
        out_type=[
            jax.ShapeDtypeStruct((_B, _DP), jnp.float32),
            jax.ShapeDtypeStruct((_B, _DP), jnp.float32),
        ],
        mesh=plsc.VectorSubcoreMesh(
            core_axis_name="c", subcore_axis_name="s",
            num_cores=_NC, num_subcores=_NS,
        ),
        scratch_types=[
            pltpu.VMEM((_BPW,), jnp.int32),
            pltpu.VMEM((_BPW // 128, 128), jnp.int32),
            pltpu.VMEM((_BPW // 128, 128), jnp.int32),
            pltpu.VMEM((_BPW,), jnp.int32),
            pltpu.VMEM((_BPW, _DP), jnp.float32),
            pltpu.SemaphoreType.DMA,
            pltpu.SemaphoreType.DMA,
        ],
    )
    return fn(slp, lgp, index, post)


_BK = 8192


def _tp_body(x_ref, o_ref):
    o_ref[:, 0:_D] = jnp.swapaxes(x_ref[...], 0, 1)


def _tc_transpad(stv):
    nblk = (_N + _BK - 1) // _BK
    return pl.pallas_call(
        _tp_body,
        grid=(nblk,),
        in_specs=[pl.BlockSpec((_D, _BK), lambda b: (0, b))],
        out_specs=pl.BlockSpec((_BK, _DP), lambda b: (b, 0)),
        out_shape=jax.ShapeDtypeStruct((_N, _DP), jnp.float32),
    )(stv)


_BT = 4096
_NB = _B // _BT
_NEG = -1e30


def _tc_loss_body(x_ref, g_ref, l2_ref, lbl_ref, ce_ref, selc_ref):
    b = pl.program_id(0)
    cols = lax.broadcasted_iota(jnp.int32, (_BT, _DP), 1)
    valid = cols < _D
    x = jnp.where(valid, x_ref[...], _NEG)
    m = jnp.max(x, axis=1, keepdims=True)
    ex = jnp.where(valid, jnp.exp(x - m), 0.0)
    s = jnp.sum(ex, axis=1, keepdims=True)
    lse = m + jnp.log(s)
    l2 = jnp.where(valid, l2_ref[...], _NEG)
    m2 = jnp.max(l2, axis=1, keepdims=True)
    e2 = jnp.where(valid, jnp.exp(l2 - m2), 0.0)
    p2 = e2 / jnp.sum(e2, axis=1, keepdims=True)
    tgt = _MOM * g_ref[...] + (1.0 - _MOM) * p2
    lpt = jnp.where(valid, (x - lse) * tgt, 0.0)
    selc_part = -jnp.sum(lpt)
    lbl = lbl_ref[0, 0, :]
    picked = jnp.sum(jnp.where(cols == lbl[:, None], x - lse, 0.0))

    @pl.when(b == 0)
    def _():
        ce_ref[...] = jnp.zeros((1, 1), jnp.float32)
        selc_ref[...] = jnp.zeros((1, 1), jnp.float32)

    ce_ref[...] += jnp.full((1, 1), -picked)
    selc_ref[...] += jnp.full((1, 1), selc_part)


def _tc_loss(lgp, g, l2, labels):
    lbl3 = labels.astype(jnp.int32).reshape(_NB, 1, _BT)
    ce_sum, selc_sum = pl.pallas_call(
        _tc_loss_body,
        grid=(_NB,),
        in_specs=[
            pl.BlockSpec((_BT, _DP), lambda b: (b, 0)),
            pl.BlockSpec((_BT, _DP), lambda b: (b, 0)),
            pl.BlockSpec((_BT, _DP), lambda b: (b, 0)),
            pl.BlockSpec((1, 1, _BT), lambda b: (b, 0, 0)),
        ],
        out_specs=[
            pl.BlockSpec((1, 1), lambda b: (0, 0)),
            pl.BlockSpec((1, 1), lambda b: (0, 0)),
        ],
        out_shape=[
            jax.ShapeDtypeStruct((1, 1), jnp.float32),
            jax.ShapeDtypeStruct((1, 1), jnp.float32),
        ],
    )(lgp, g, l2, lbl3)
    return ce_sum[0, 0], selc_sum[0, 0]


def kernel(logits, soft_labels, labels, index, epoch):
    slp = _tc_transpad(jnp.swapaxes(soft_labels, 0, 1))
    lgp = jnp.pad(logits, ((0, 0), (0, _DP - _D)))
    pos = jnp.arange(_B, dtype=jnp.int32)
    idx32 = index.astype(jnp.int32)
    post = _sc_scatter(idx32, pos)
    g, l2 = _sc_gather(slp, lgp, idx32, post)
    ce_sum, selc_sum = _tc_loss(lgp, g, l2, labels)
    ce = ce_sum / _B
    selc = selc_sum / _B
    return jnp.where(epoch <= _ES, ce, selc)


# speedup vs baseline: 1.0331x; 1.0331x over previous
import functools

import jax
import jax.numpy as jnp
from jax import lax
from jax.experimental import pallas as pl
from jax.experimental.pallas import tpu as pltpu
from jax.experimental.pallas import tpu_sc as plsc

_B = 16384
_D = 100
_DP = 128
_N = 1000000
_NC = 2
_NS = 16
_NW = _NC * _NS
_BPW = _B // _NW
_SPT = _B // _NS
_ES = 10
_MOM = 0.9


def _sc_scatter_body(idx_hbm, pos_hbm, post, sidx2, spos2, sem_s):
    c = lax.axis_index("c")
    s = lax.axis_index("s")
    wid = s * _NC + c
    base = wid * _BPW
    for k in range(_BPW // 128):
        pltpu.sync_copy(idx_hbm.at[pl.ds(base + k * 128, 128)],
                        sidx2.at[k])
        pltpu.sync_copy(pos_hbm.at[pl.ds(base + k * 128, 128)],
                        spos2.at[k])
    scps = [
        pltpu.async_copy(spos2.at[k], post.at[sidx2.at[k]], sem_s)
        for k in range(_BPW // 128)
    ]
    for cp in scps:
        cp.wait()


def _sc_scatter(index, pos):
    fn = pl.kernel(
        _sc_scatter_body,
        out_type=jax.ShapeDtypeStruct((_N,), jnp.int32),
        mesh=plsc.VectorSubcoreMesh(
            core_axis_name="c", subcore_axis_name="s",
            num_cores=_NC, num_subcores=_NS,
        ),
        scratch_types=[
            pltpu.VMEM((_BPW // 128, 128), jnp.int32),
            pltpu.VMEM((_BPW // 128, 128), jnp.int32),
            pltpu.SemaphoreType.DMA,
        ],
    )
    return fn(index, pos)


def _sc_body(slp, lgp, idx_hbm, post,
             g_out, l2_out,
             idx_v, oidx2, w2, w_v, rowbuf,
             sem_s, sem_g):
    c = lax.axis_index("c")
    s = lax.axis_index("s")
    wid = s * _NC + c
    base = wid * _BPW

    pltpu.sync_copy(idx_hbm.at[pl.ds(base, _BPW)], idx_v)

    gcp = pltpu.async_copy(slp.at[idx_v], rowbuf, sem_g)

    for k in range(_BPW // 128):
        pltpu.sync_copy(idx_hbm.at[pl.ds(base + k * 128, 128)],
                        oidx2.at[k])
    wcps = [
        pltpu.async_copy(post.at[oidx2.at[k]], w2.at[k], sem_s)
        for k in range(_BPW // 128)
    ]
    for cp in wcps:
        cp.wait()
    for k in range(_BPW // 128):
        for j in range(8):
            v = w2[k, pl.ds(j * 16, 16)]
            w_v[pl.ds(k * 128 + j * 16, 16)] = jnp.clip(v, 0, _B - 1)

    gcp.wait()
    pltpu.sync_copy(rowbuf, g_out.at[pl.ds(base, _BPW)])

    pltpu.async_copy(lgp.at[w_v], rowbuf, sem_g).wait()
    pltpu.sync_copy(rowbuf, l2_out.at[pl.ds(base, _BPW)])


def _sc_gather(slp, lgp, index, post):
    fn = pl.kernel(
        _sc_body,
        out_type=[
            jax.ShapeDtypeStruct((_B, _DP), jnp.float32),
            jax.ShapeDtypeStruct((_B, _DP), jnp.float32),
        ],
        mesh=plsc.VectorSubcoreMesh(
            core_axis_name="c", subcore_axis_name="s",
            num_cores=_NC, num_subcores=_NS,
        ),
        scratch_types=[
            pltpu.VMEM((_BPW,), jnp.int32),
            pltpu.VMEM((_BPW // 128, 128), jnp.int32),
            pltpu.VMEM((_BPW // 128, 128), jnp.int32),
            pltpu.VMEM((_BPW,), jnp.int32),
            pltpu.VMEM((_BPW, _DP), jnp.float32),
            pltpu.SemaphoreType.DMA,
            pltpu.SemaphoreType.DMA,
        ],
    )
    return fn(slp, lgp, index, post)


_BK = 16384


def _tp_body(x_ref, o_ref):
    o_ref[:, 0:_D] = jnp.swapaxes(x_ref[...], 0, 1)


def _tc_transpad(stv):
    nblk = (_N + _BK - 1) // _BK
    return pl.pallas_call(
        _tp_body,
        grid=(nblk,),
        in_specs=[pl.BlockSpec((_D, _BK), lambda b: (0, b))],
        out_specs=pl.BlockSpec((_BK, _DP), lambda b: (b, 0)),
        out_shape=jax.ShapeDtypeStruct((_N, _DP), jnp.float32),
    )(stv)


_BT = 4096
_NB = _B // _BT
_NEG = -1e30


def _tc_loss_body(x_ref, g_ref, l2_ref, lbl_ref, ce_ref, selc_ref):
    b = pl.program_id(0)
    cols = lax.broadcasted_iota(jnp.int32, (_BT, _DP), 1)
    valid = cols < _D
    x = jnp.where(valid, x_ref[...], _NEG)
    m = jnp.max(x, axis=1, keepdims=True)
    ex = jnp.where(valid, jnp.exp(x - m), 0.0)
    s = jnp.sum(ex, axis=1, keepdims=True)
    lse = m + jnp.log(s)
    l2 = jnp.where(valid, l2_ref[...], _NEG)
    m2 = jnp.max(l2, axis=1, keepdims=True)
    e2 = jnp.where(valid, jnp.exp(l2 - m2), 0.0)
    p2 = e2 / jnp.sum(e2, axis=1, keepdims=True)
    tgt = _MOM * g_ref[...] + (1.0 - _MOM) * p2
    lpt = jnp.where(valid, (x - lse) * tgt, 0.0)
    selc_part = -jnp.sum(lpt)
    lbl = lbl_ref[0, 0, :]
    picked = jnp.sum(jnp.where(cols == lbl[:, None], x - lse, 0.0))

    @pl.when(b == 0)
    def _():
        ce_ref[...] = jnp.zeros((1, 1), jnp.float32)
        selc_ref[...] = jnp.zeros((1, 1), jnp.float32)

    ce_ref[...] += jnp.full((1, 1), -picked)
    selc_ref[...] += jnp.full((1, 1), selc_part)


def _tc_loss(lgp, g, l2, labels):
    lbl3 = labels.astype(jnp.int32).reshape(_NB, 1, _BT)
    ce_sum, selc_sum = pl.pallas_call(
        _tc_loss_body,
        grid=(_NB,),
        in_specs=[
            pl.BlockSpec((_BT, _DP), lambda b: (b, 0)),
            pl.BlockSpec((_BT, _DP), lambda b: (b, 0)),
            pl.BlockSpec((_BT, _DP), lambda b: (b, 0)),
            pl.BlockSpec((1, 1, _BT), lambda b: (b, 0, 0)),
        ],
        out_specs=[
            pl.BlockSpec((1, 1), lambda b: (0, 0)),
            pl.BlockSpec((1, 1), lambda b: (0, 0)),
        ],
        out_shape=[
            jax.ShapeDtypeStruct((1, 1), jnp.float32),
            jax.ShapeDtypeStruct((1, 1), jnp.float32),
        ],
    )(lgp, g, l2, lbl3)
    return ce_sum[0, 0], selc_sum[0, 0]


def kernel(logits, soft_labels, labels, index, epoch):
    slp = _tc_transpad(jnp.swapaxes(soft_labels, 0, 1))
    lgp = jnp.pad(logits, ((0, 0), (0, _DP - _D)))
    pos = jnp.arange(_B, dtype=jnp.int32)
    idx32 = index.astype(jnp.int32)
    post = _sc_scatter(idx32, pos)
    g, l2 = _sc_gather(slp, lgp, idx32, post)
    ce_sum, selc_sum = _tc_loss(lgp, g, l2, labels)
    ce = ce_sum / _B
    selc = selc_sum / _B
    return jnp.where(epoch <= _ES, ce, selc)
